# baseline (speedup 1.0000x reference)
import jax
import jax.numpy as jnp
from jax.experimental import pallas as pl
from functools import partial

C_CAT = 24
LAT = C_CAT * C_CAT
HID = 1024
SIDE = 64
D_STATE = 16
D_CONV = 4
N_LAYERS = 8

PREC = jax.lax.Precision.HIGHEST


def _silu(x):
    return x * jax.nn.sigmoid(x)


def _mamba(p, x):
    Bb, S, D = x.shape
    xz = jnp.einsum('bsd,de->bse', x, p['in_W'], precision=PREC)
    di = xz.shape[-1] // 2
    xr, z = xz[..., :di], xz[..., di:]
    xp = jnp.pad(xr, ((0, 0), (D_CONV - 1, 0), (0, 0)))
    xc = sum(xp[:, k:k + S, :] * p['conv_W'][:, k] for k in range(D_CONV)) + p['conv_b']
    xc = _silu(xc)
    proj = jnp.einsum('bsd,de->bse', xc, p['x_W'], precision=PREC)
    dtr = p['dt_W'].shape[0]
    dt = jax.nn.softplus(jnp.einsum('bsr,rd->bsd', proj[..., :dtr], p['dt_W'], precision=PREC) + p['dt_b'])
    Bc = proj[..., dtr:dtr + D_STATE]
    Cc = proj[..., dtr + D_STATE:]
    A = -jnp.exp(p['A_log'])
    dA = jnp.exp(dt[..., None] * A)
    dBx = dt[..., None] * Bc[:, :, None, :] * xc[..., None]

    def step(h, inp):
        a, bx = inp
        h = a * h + bx
        return h, h

    h0 = jnp.zeros((Bb, di, D_STATE), x.dtype)
    _, hs = jax.lax.scan(step, h0, (jnp.moveaxis(dA, 1, 0), jnp.moveaxis(dBx, 1, 0)))
    y = jnp.einsum('sbdn,bsn->bsd', hs, Cc, precision=PREC) + xc * p['D']
    y = y * _silu(z)
    return jnp.einsum('bsd,de->bse', y, p['out_W'], precision=PREC)


def _vae_decode(p, zlat):
    x = jax.nn.relu(jnp.einsum('nl,lf->nf', zlat, p['fc_W'], precision=PREC) + p['fc_b'])
    x = x.reshape(-1, 64, 8, 8)
    n_dec = len(p['dec_W'])
    for i in range(n_dec):
        x = jax.lax.conv_transpose(x, p['dec_W'][i], strides=(2, 2), padding='SAME',
                                   dimension_numbers=('NCHW', 'OIHW', 'NCHW'),
                                   precision=PREC)
        x = x + p['dec_b'][i][None, :, None, None]
        x = jax.nn.sigmoid(x) if i == n_dec - 1 else jax.nn.relu(x)
    return x.reshape(-1, SIDE, SIDE)


def _st_sample(probs, key):
    g = jax.random.gumbel(key, probs.shape, probs.dtype)
    idx = jnp.argmax(jnp.log(probs) + g, axis=-1)
    onehot = jax.nn.one_hot(idx, probs.shape[-1], dtype=probs.dtype)
    return onehot + probs - jax.lax.stop_gradient(probs)


def _pallas_copy(x):
    def _k(x_ref, o_ref):
        o_ref[...] = x_ref[...]
    return pl.pallas_call(
        _k, out_shape=jax.ShapeDtypeStruct(x.shape, x.dtype), name="probe_copy")(x)


def kernel(obs_lats, actions, rewards, params):
    B, S, _ = obs_lats.shape
    states = jnp.concatenate([obs_lats, actions[..., None], rewards], axis=-1)
    pad = HID - states.shape[-1]
    states = jnp.concatenate([states, jnp.zeros((B, S, pad), states.dtype)], axis=-1)
    h = states
    for p in params['predictor']:
        h = h + _mamba(p, h)
    t = _mamba(params['img_head_mamba'], h)
    logits = jnp.einsum('bsd,dl->bsl', t, params['img_W'], precision=PREC) + params['img_b']
    probs = jax.nn.softmax(logits.reshape(B, S, C_CAT, C_CAT), axis=-1)
    probs = 0.99 * probs + 0.01 / C_CAT
    pred_dists = probs.reshape(B, S, LAT)
    p_flat = probs.reshape(B * S * C_CAT, C_CAT)
    samples = _st_sample(p_flat, jax.random.key(42)).reshape(B * S, LAT)
    pred_obs = _vae_decode(params['vae'], samples).reshape(B, S, SIDE, SIDE)
    r = _mamba(params['reward_mamba'], h)
    pred_rewards = jnp.einsum('bsd,do->bso', r, params['rew_W'], precision=PREC) + params['rew_b']
    pred_obs = _pallas_copy(pred_obs)
    return pred_obs, pred_dists, pred_rewards, h


# speedup vs baseline: 3.4677x; 3.4677x over previous
import jax
import jax.numpy as jnp
from jax.experimental import pallas as pl
from jax.experimental.pallas import tpu as pltpu

C_CAT = 24
LAT = C_CAT * C_CAT
HID = 1024
SIDE = 64
DS = 16
DCONV = 4
NL = 8
DI = 2 * HID
NC = 2
DH = DI // NC
DTR = HID // 16
B, S = 2, 64
BS = B * S
FC = 64 * 8 * 8

f32 = jnp.float32
_INTERPRET = False

_CPARAMS = pltpu.CompilerParams(
    dimension_semantics=("parallel",),
    vmem_limit_bytes=56 * 1024 * 1024,
)


def _silu(x):
    return x * jax.nn.sigmoid(x)


def _dot(a, b):
    return jnp.dot(a, b, preferred_element_type=f32)


def _mix(x, wx_ref, wz_ref, cw_ref, cb_ref, xw_ref):
    xb = x.reshape(BS, HID)
    xr = _dot(xb, wx_ref[...]).reshape(B, S, DH)
    z = _dot(xb, wz_ref[...]).reshape(B, S, DH)
    acc = jnp.broadcast_to(cb_ref[0], (B, S, DH))
    for k in range(DCONV):
        sh = DCONV - 1 - k
        if sh == 0:
            xs = xr
        else:
            xs = jnp.concatenate(
                [jnp.zeros((B, sh, DH), f32), xr[:, :S - sh]], axis=1)
        acc = acc + xs * cw_ref[:, k].reshape(1, 1, DH)
    xc = _silu(acc)
    pj = _dot(xc.reshape(BS, DH), xw_ref[...]).reshape(B, S, 96)
    return xc, _silu(z), pj


def _pa_body(h_ref, pp_ref, wx_ref, wz_ref, cw_ref, cb_ref, xw_ref,
             hout_ref, xc_ref, g_ref, pj_ref):
    c = pl.program_id(0)
    x = h_ref[...] + pp_ref[0] + pp_ref[1]
    hout_ref[...] = jnp.where(c == 0, x[0], x[1])[None]
    xc, g, pj = _mix(x, wx_ref, wz_ref, cw_ref, cb_ref, xw_ref)
    xc_ref[0] = xc
    g_ref[0] = g
    pj_ref[0] = pj


def _phase_a(h, pp, in_w, conv_w, cb, x_w):
    full = lambda a: pl.BlockSpec(a.shape, lambda c: (0,) * a.ndim)
    outs = (
        jax.ShapeDtypeStruct((B, S, HID), f32),
        jax.ShapeDtypeStruct((NC, B, S, DH), f32),
        jax.ShapeDtypeStruct((NC, B, S, DH), f32),
        jax.ShapeDtypeStruct((NC, B, S, 96), f32),
    )
    return pl.pallas_call(
        _pa_body,
        grid=(NC,),
        in_specs=[
            full(h), full(pp),
            pl.BlockSpec((HID, DH), lambda c: (0, c)),
            pl.BlockSpec((HID, DH), lambda c: (0, NC + c)),
            pl.BlockSpec((DH, DCONV), lambda c: (c, 0)),
            pl.BlockSpec((1, 1, DH), lambda c: (c, 0, 0)),
            pl.BlockSpec((DH, 96), lambda c: (c, 0)),
        ],
        out_specs=[
            pl.BlockSpec((1, S, HID), lambda c: (c, 0, 0)),
            pl.BlockSpec((1, B, S, DH), lambda c: (c, 0, 0, 0)),
            pl.BlockSpec((1, B, S, DH), lambda c: (c, 0, 0, 0)),
            pl.BlockSpec((1, B, S, 96), lambda c: (c, 0, 0, 0)),
        ],
        out_shape=outs,
        compiler_params=_CPARAMS,
        interpret=_INTERPRET,
        name="mamba_phase_a",
    )(h, pp, in_w, in_w, conv_w, cb, x_w)


def _ssm_y(xc, g, pj, dw_ref, db_ref, at_ref, dv_ref):
    dti = pj[..., :DTR]
    Bc = pj[..., DTR:DTR + DS]
    Cc = pj[..., DTR + DS:]
    dtl = _dot(dti.reshape(BS, DTR), dw_ref[...]).reshape(B, S, DH)
    dt = jax.nn.softplus(dtl + db_ref[0])
    dts = jnp.moveaxis(dt, 1, 0)
    xcs = jnp.moveaxis(xc, 1, 0)
    Bcs = jnp.moveaxis(Bc, 1, 0)
    Ccs = jnp.moveaxis(Cc, 1, 0)
    at = at_ref[0]
    dAs = jnp.exp(dts[:, :, None, :] * at[None, None])
    dBs = (dts * xcs)[:, :, None, :] * Bcs[..., None]
    h = jnp.zeros((B, DS, DH), f32)
    ys = []
    for t in range(S):
        h = dAs[t] * h + dBs[t]
        ys.append(jnp.sum(h * Ccs[t][:, :, None], axis=1))
    y = jnp.stack(ys, axis=1)
    y = y + dv_ref[0] * xc
    return y * g


def _pb_body(xc_ref, g_ref, pj_ref, dw_ref, db_ref, at_ref, dv_ref, ow_ref,
             op_ref):
    pj = pj_ref[0] + pj_ref[1]
    y = _ssm_y(xc_ref[0], g_ref[0], pj, dw_ref, db_ref, at_ref, dv_ref)
    op = _dot(y.reshape(BS, DH), ow_ref[...])
    op_ref[0] = op.reshape(B, S, HID)


def _phase_b(xc, g, pj, dt_w, db, at, dv, out_w):
    full = lambda a: pl.BlockSpec(a.shape, lambda c: (0,) * a.ndim)
    return pl.pallas_call(
        _pb_body,
        grid=(NC,),
        in_specs=[
            pl.BlockSpec((1, B, S, DH), lambda c: (c, 0, 0, 0)),
            pl.BlockSpec((1, B, S, DH), lambda c: (c, 0, 0, 0)),
            full(pj),
            pl.BlockSpec((DTR, DH), lambda c: (0, c)),
            pl.BlockSpec((1, 1, DH), lambda c: (c, 0, 0)),
            pl.BlockSpec((1, DS, DH), lambda c: (c, 0, 0)),
            pl.BlockSpec((1, 1, DH), lambda c: (c, 0, 0)),
            pl.BlockSpec((DH, HID), lambda c: (c, 0)),
        ],
        out_specs=[pl.BlockSpec((1, B, S, HID), lambda c: (c, 0, 0, 0))],
        out_shape=[jax.ShapeDtypeStruct((NC, B, S, HID), f32)],
        compiler_params=_CPARAMS,
        interpret=_INTERPRET,
        name="mamba_phase_b",
    )(xc, g, pj, dt_w, db, at, dv, out_w)[0]


def _t1_body(h_ref, pp_ref,
             wxi_ref, wzi_ref, cwi_ref, cbi_ref, xwi_ref,
             wxr_ref, wzr_ref, cwr_ref, cbr_ref, xwr_ref,
             hout_ref, xci_ref, gi_ref, pji_ref, xcr_ref, gr_ref, pjr_ref):
    c = pl.program_id(0)
    x = h_ref[...] + pp_ref[0] + pp_ref[1]
    hout_ref[...] = jnp.where(c == 0, x[0], x[1])[None]
    xc, g, pj = _mix(x, wxi_ref, wzi_ref, cwi_ref, cbi_ref, xwi_ref)
    xci_ref[0], gi_ref[0], pji_ref[0] = xc, g, pj
    xc, g, pj = _mix(x, wxr_ref, wzr_ref, cwr_ref, cbr_ref, xwr_ref)
    xcr_ref[0], gr_ref[0], pjr_ref[0] = xc, g, pj


def _tail_double_a(h, pp, wi, wr):
    full = lambda a: pl.BlockSpec(a.shape, lambda c: (0,) * a.ndim)
    mspecs = [
        pl.BlockSpec((HID, DH), lambda c: (0, c)),
        pl.BlockSpec((HID, DH), lambda c: (0, NC + c)),
        pl.BlockSpec((DH, DCONV), lambda c: (c, 0)),
        pl.BlockSpec((1, 1, DH), lambda c: (c, 0, 0)),
        pl.BlockSpec((DH, 96), lambda c: (c, 0)),
    ]
    souts = [
        pl.BlockSpec((1, B, S, DH), lambda c: (c, 0, 0, 0)),
        pl.BlockSpec((1, B, S, DH), lambda c: (c, 0, 0, 0)),
        pl.BlockSpec((1, B, S, 96), lambda c: (c, 0, 0, 0)),
    ]
    shapes = [
        jax.ShapeDtypeStruct((NC, B, S, DH), f32),
        jax.ShapeDtypeStruct((NC, B, S, DH), f32),
        jax.ShapeDtypeStruct((NC, B, S, 96), f32),
    ]
    return pl.pallas_call(
        _t1_body,
        grid=(NC,),
        in_specs=[full(h), full(pp)] + mspecs + mspecs,
        out_specs=[pl.BlockSpec((1, S, HID), lambda c: (c, 0, 0))]
                  + souts + souts,
        out_shape=[jax.ShapeDtypeStruct((B, S, HID), f32)] + shapes + shapes,
        compiler_params=_CPARAMS,
        interpret=_INTERPRET,
        name="tail_double_a",
    )(h, pp,
      wi['in'], wi['in'], wi['cw'], wi['cb'], wi['xw'],
      wr['in'], wr['in'], wr['cw'], wr['cb'], wr['xw'])


def _t3_body(tp_ref, rp_ref, wimg_ref, bimg_ref, gum_ref, reww_ref, rb_ref,
             pd_ref, sm_ref, pr_ref):
    c = pl.program_id(0)
    t = (tp_ref[0] + tp_ref[1]).reshape(BS, HID)
    wimg = jnp.where(c == 0, wimg_ref[:, :LAT // 2], wimg_ref[:, LAT // 2:])
    lg = _dot(t, wimg) + bimg_ref[0]
    l4 = lg.reshape(BS, LAT // (2 * C_CAT), C_CAT)
    m = jnp.max(l4, axis=-1, keepdims=True)
    e = jnp.exp(l4 - m)
    p = e / jnp.sum(e, axis=-1, keepdims=True)
    p = 0.99 * p + 0.01 / C_CAT
    pd_ref[0] = p.reshape(B, S, LAT // 2)
    yv = jnp.log(p) + gum_ref[0].reshape(BS, LAT // (2 * C_CAT), C_CAT)
    my = jnp.max(yv, axis=-1, keepdims=True)
    io = jax.lax.broadcasted_iota(jnp.int32, yv.shape, 2)
    idx = jnp.min(jnp.where(yv == my, io, C_CAT), axis=-1, keepdims=True)
    sm_ref[0] = (io == idx).astype(f32).reshape(B, S, LAT // 2)
    r = (rp_ref[0] + rp_ref[1]).reshape(BS, HID)
    rw = _dot(r, reww_ref[...]).reshape(B, S, 1) + rb_ref[0, 0]
    pr_ref[...] = jnp.where(c == 0, rw[0], rw[1])[None]


def _head(tp, rp, img_w, bimg, gum, rew_w, rb):
    full = lambda a: pl.BlockSpec(a.shape, lambda c: (0,) * a.ndim)
    H2 = LAT // 2
    return pl.pallas_call(
        _t3_body,
        grid=(NC,),
        in_specs=[
            full(tp), full(rp),
            full(img_w),
            pl.BlockSpec((1, 1, H2), lambda c: (c, 0, 0)),
            pl.BlockSpec((1, B, S, H2), lambda c: (c, 0, 0, 0)),
            full(rew_w), full(rb),
        ],
        out_specs=[
            pl.BlockSpec((1, B, S, H2), lambda c: (c, 0, 0, 0)),
            pl.BlockSpec((1, B, S, H2), lambda c: (c, 0, 0, 0)),
            pl.BlockSpec((1, S, 1), lambda c: (c, 0, 0)),
        ],
        out_shape=[
            jax.ShapeDtypeStruct((NC, B, S, LAT // 2), f32),
            jax.ShapeDtypeStruct((NC, B, S, LAT // 2), f32),
            jax.ShapeDtypeStruct((B, S, 1), f32),
        ],
        compiler_params=_CPARAMS,
        interpret=_INTERPRET,
        name="head_sample",
    )(tp, rp, img_w, bimg, gum, rew_w, rb)


def _shift(x, d, axis_pair):
    if d == 0:
        return x
    pax, bax = axis_pair
    P = x.shape[pax]
    idx = [slice(None)] * x.ndim

    def base_shift(v, dd):
        bi = [slice(None)] * v.ndim
        z = list(v.shape)
        z[bax] = 1
        zero = jnp.zeros(z, v.dtype)
        if dd == 1:
            bi[bax] = slice(1, None)
            return jnp.concatenate([v[tuple(bi)], zero], axis=bax)
        bi[bax] = slice(0, v.shape[bax] - 1)
        return jnp.concatenate([zero, v[tuple(bi)]], axis=bax)

    if P == 1:
        return base_shift(x, d)
    if d == 1:
        idx[pax] = slice(1, None)
        hi = x[tuple(idx)]
        idx[pax] = slice(0, 1)
        lo = base_shift(x[tuple(idx)], 1)
        return jnp.concatenate([hi, lo], axis=pax)
    idx[pax] = slice(0, P - 1)
    lo = x[tuple(idx)]
    idx[pax] = slice(P - 1, P)
    hi = base_shift(x[tuple(idx)], -1)
    return jnp.concatenate([hi, lo], axis=pax)


def _deconv(x, tw_ref, last):
    N, PY, PX, H, W, Ci = x.shape
    Co = tw_ref.shape[-1]
    M = N * PY * PX * H * W
    ph = []
    for a in (0, 1):
        for b in (0, 1):
            acc = jnp.zeros((M // W, Co), f32)
            for dy in (0, 1):
                for dx in (0, 1):
                    win = _shift(x, a + dy - 1, (1, 3))
                    win = _shift(win, b + dx - 1, (2, 4))
                    tap = tw_ref[(a * 2 + b) * 4 + dy * 2 + dx]
                    acc = acc + _dot(win.reshape(M // W, W * Ci), tap)
            if last:
                ph.append(acc.reshape(N, PY, PX, H, W))
            else:
                ph.append(acc.reshape(N, PY, PX, H, W, Co // W))
    row0 = jnp.stack([ph[0], ph[1]], axis=3)
    row1 = jnp.stack([ph[2], ph[3]], axis=3)
    out = jnp.stack([row0, row1], axis=2)
    if last:
        out = out.reshape(N, 2 * PY, 2 * PX, H, W)
        return jax.nn.sigmoid(out)
    out = out.reshape(N, 2 * PY, 2 * PX, H, W, Co // W)
    return jnp.maximum(out, 0.0)


CH = 8


def _t4_body(sm_ref, fcw_ref, fcb_ref, w1_ref, w2_ref, w3_ref, po_ref):
    s = jnp.concatenate([sm_ref[0, 0], sm_ref[1, 0]], axis=-1)
    x = _dot(s, fcw_ref[...]) + fcb_ref[...]
    x = jnp.maximum(x, 0.0).reshape(CH, 1, 1, 8, 8, 64)
    x = _deconv(x, w1_ref, last=False)
    x = _deconv(x, w2_ref, last=False)
    x = _deconv(x, w3_ref, last=True)
    po_ref[0] = x


def _vae(sm, fcw, fcb, w1, w2, w3):
    full = lambda a: pl.BlockSpec(a.shape, lambda c: (0,) * a.ndim)
    full = lambda a: pl.BlockSpec(a.shape, lambda c, g: (0,) * a.ndim)
    return pl.pallas_call(
        _t4_body,
        grid=(NC, S // CH),
        in_specs=[
            pl.BlockSpec((NC, 1, CH, LAT // 2), lambda c, g: (0, c, g, 0)),
            full(fcw), full(fcb), full(w1), full(w2), full(w3),
        ],
        out_specs=[pl.BlockSpec((1, CH, 8, 8, 8, 8),
                                lambda c, g: (c, g, 0, 0, 0, 0))],
        out_shape=[jax.ShapeDtypeStruct((B, S, 8, 8, 8, 8), f32)],
        compiler_params=pltpu.CompilerParams(
            dimension_semantics=("parallel", "arbitrary"),
            vmem_limit_bytes=56 * 1024 * 1024,
        ),
        interpret=_INTERPRET,
        name="vae_decode",
    )(sm, fcw, fcb, w1, w2, w3)[0]


def _prep_small(p):
    return dict(
        cb=p['conv_b'].reshape(NC, 1, DH),
        db=p['dt_b'].reshape(NC, 1, DH),
        at=jnp.transpose(-jnp.exp(p['A_log'])).reshape(DS, NC, DH)
            .transpose(1, 0, 2),
        dv=p['D'].reshape(NC, 1, DH),
    )


def _mamba_pallas(h, pp, p, sp):
    hout, xc, g, pj = _phase_a(h, pp, p['in_W'], p['conv_W'], sp['cb'],
                               p['x_W'])
    op = _phase_b(xc, g, pj, p['dt_W'], sp['db'], sp['at'], sp['dv'],
                  p['out_W'])
    return hout, op


def _deconv_taps(W):
    taps = []
    for a in (0, 1):
        for b in (0, 1):
            for dy in (0, 1):
                for dx in (0, 1):
                    t = jnp.transpose(W[:, :, 2 * dy + a, 2 * dx + b])
                    taps.append(jnp.kron(jnp.eye(8, dtype=f32), t))
    return jnp.stack(taps)


def kernel(obs_lats, actions, rewards, params):
    states = jnp.concatenate([obs_lats, actions[..., None], rewards], axis=-1)
    states = jnp.concatenate(
        [states, jnp.zeros((B, S, HID - states.shape[-1]), f32)], axis=-1)

    h = states
    pp = jnp.zeros((NC, B, S, HID), f32)
    for p in params['predictor']:
        h, pp = _mamba_pallas(h, pp, p, _prep_small(p))

    pi, pr_ = params['img_head_mamba'], params['reward_mamba']
    spi, spr = _prep_small(pi), _prep_small(pr_)
    wi = dict(**{'in': pi['in_W']}, cw=pi['conv_W'], cb=spi['cb'],
              xw=pi['x_W'])
    wr = dict(**{'in': pr_['in_W']}, cw=pr_['conv_W'], cb=spr['cb'],
              xw=pr_['x_W'])
    (h_fin, xci, gi, pji, xcr, gr, pjr) = _tail_double_a(h, pp, wi, wr)
    tp = _phase_b(xci, gi, pji, pi['dt_W'], spi['db'], spi['at'], spi['dv'],
                  pi['out_W'])
    rp = _phase_b(xcr, gr, pjr, pr_['dt_W'], spr['db'], spr['at'], spr['dv'],
                  pr_['out_W'])

    gum = jax.random.gumbel(jax.random.key(42), (BS * C_CAT, C_CAT), f32)
    gum = gum.reshape(B, S, LAT)
    gum = jnp.stack([gum[..., :LAT // 2], gum[..., LAT // 2:]])
    bimg = params['img_b'].reshape(NC, 1, LAT // 2)
    rb = params['rew_b'].reshape(1, 1)
    pd, samples, pred_rewards = _head(
        tp, rp, params['img_W'], bimg, gum, params['rew_W'], rb)
    pred_dists = jnp.moveaxis(pd, 0, 2).reshape(B, S, LAT)

    v = params['vae']
    fcw = v['fc_W'].reshape(LAT, 64, 8, 8).transpose(0, 2, 3, 1)
    fcw = fcw.reshape(LAT, FC)
    fcb = v['fc_b'].reshape(64, 8, 8).transpose(1, 2, 0).reshape(1, FC)
    w1 = _deconv_taps(v['dec_W'][0])
    w2 = _deconv_taps(v['dec_W'][1])
    w3 = _deconv_taps(v['dec_W'][2])
    po = _vae(samples, fcw, fcb, w1, w2, w3)
    pred_obs = jnp.transpose(po, (0, 1, 4, 2, 5, 3)).reshape(B, S, SIDE, SIDE)

    return pred_obs, pred_dists, pred_rewards, h_fin
